# Initial kernel scaffold; baseline (speedup 1.0000x reference)
import jax
import jax.numpy as jnp
from jax.experimental import pallas as pl


def kernel(x, Wg, w1, w2, w3):
    raise NotImplementedError("write your pallas kernel here")


# speedup vs baseline: 2.4960x; 2.4960x over previous
import functools

import jax
import jax.numpy as jnp
from jax import lax
from jax.experimental import pallas as pl
from jax.experimental.pallas import tpu as pltpu
from jax.experimental.pallas import tpu_sc as plsc

E = 64
K = 2
D = 768
F = 512
T = 4096
BM = 128
G = 128
PAD_CAP = G * BM
TB = 512


def _gate_body(x_ref, wg_ref, w_ref, e_ref):
    logits = jnp.dot(x_ref[...], wg_ref[...], preferred_element_type=jnp.float32)
    iota = lax.broadcasted_iota(jnp.int32, logits.shape, 1)
    m1 = jnp.max(logits, axis=1, keepdims=True)
    e1 = jnp.min(jnp.where(logits == m1, iota, E), axis=1, keepdims=True)
    masked = jnp.where(iota == e1, -jnp.inf, logits)
    m2 = jnp.max(masked, axis=1, keepdims=True)
    e2 = jnp.min(jnp.where(masked == m2, iota, E), axis=1, keepdims=True)
    z = jnp.exp(m2 - m1)
    denom = 1.0 + z
    w_ref[...] = jnp.concatenate([1.0 / denom, z / denom], axis=1)
    e_ref[...] = jnp.concatenate([e1, e2], axis=1)


def _gating(x, Wg):
    return pl.pallas_call(
        _gate_body,
        grid=(T // TB,),
        in_specs=[
            pl.BlockSpec((TB, D), lambda i: (i, 0)),
            pl.BlockSpec((D, E), lambda i: (0, 0)),
        ],
        out_specs=[
            pl.BlockSpec((TB, K), lambda i: (i, 0)),
            pl.BlockSpec((TB, K), lambda i: (i, 0)),
        ],
        out_shape=[
            jax.ShapeDtypeStruct((T, K), jnp.float32),
            jax.ShapeDtypeStruct((T, K), jnp.int32),
        ],
    )(x, Wg)


def _sc_gather(table, idx, n_rows):
    info = plsc.get_sparse_core_info()
    nw = info.num_cores * info.num_subcores
    per_w = n_rows // nw
    ch = min(per_w, 128)
    n_ch = per_w // ch
    mesh = plsc.VectorSubcoreMesh(core_axis_name="c", subcore_axis_name="s")

    @functools.partial(
        pl.kernel,
        out_type=jax.ShapeDtypeStruct((n_rows, D), jnp.float32),
        mesh=mesh,
        scratch_types=[
            pltpu.VMEM((ch,), jnp.int32),
            pltpu.VMEM((ch, D), jnp.float32),
            pltpu.SemaphoreType.DMA,
        ],
    )
    def k(table_hbm, idx_hbm, out_hbm, idx_v, rows_v, sem):
        wid = lax.axis_index("s") * info.num_cores + lax.axis_index("c")
        for j in range(n_ch):
            base = wid * per_w + j * ch
            pltpu.sync_copy(idx_hbm.at[pl.ds(base, ch)], idx_v)
            pltpu.async_copy(table_hbm.at[idx_v], rows_v, sem).wait()
            pltpu.sync_copy(rows_v, out_hbm.at[pl.ds(base, ch)])

    return k(table, idx)


def _gmm_body(be_ref, na_ref, x_ref, w1_ref, w2_ref, w3_ref, o_ref):
    i = pl.program_id(0)

    @pl.when(i < na_ref[0])
    def _compute():
        xb = x_ref[...]
        a = lax.dot_general(xb, w1_ref[0], (((1,), (1,)), ((), ())),
                            preferred_element_type=jnp.float32)
        b = lax.dot_general(xb, w3_ref[0], (((1,), (1,)), ((), ())),
                            preferred_element_type=jnp.float32)
        h = a * jax.nn.sigmoid(a) * b
        o_ref[...] = lax.dot_general(h, w2_ref[0], (((1,), (1,)), ((), ())),
                                     preferred_element_type=jnp.float32)

    @pl.when(i >= na_ref[0])
    def _skip():
        o_ref[...] = jnp.zeros_like(o_ref)


def _gmm(x_sorted, w1, w2, w3, block_expert, num_active):
    grid_spec = pltpu.PrefetchScalarGridSpec(
        num_scalar_prefetch=2,
        grid=(G,),
        in_specs=[
            pl.BlockSpec((BM, D), lambda i, be, na: (i, 0)),
            pl.BlockSpec((1, F, D), lambda i, be, na: (be[i], 0, 0)),
            pl.BlockSpec((1, D, F), lambda i, be, na: (be[i], 0, 0)),
            pl.BlockSpec((1, F, D), lambda i, be, na: (be[i], 0, 0)),
        ],
        out_specs=pl.BlockSpec((BM, D), lambda i, be, na: (i, 0)),
    )
    return pl.pallas_call(
        _gmm_body,
        grid_spec=grid_spec,
        out_shape=jax.ShapeDtypeStruct((PAD_CAP, D), jnp.float32),
    )(block_expert, num_active, x_sorted, w1, w2, w3)


def _combine_body(w_ref, z0_ref, z1_ref, o_ref):
    o_ref[...] = w_ref[:, 0:1] * z0_ref[...] + w_ref[:, 1:2] * z1_ref[...]


def _combine(wts, z0, z1):
    return pl.pallas_call(
        _combine_body,
        grid=(T // TB,),
        in_specs=[
            pl.BlockSpec((TB, K), lambda i: (i, 0)),
            pl.BlockSpec((TB, D), lambda i: (i, 0)),
            pl.BlockSpec((TB, D), lambda i: (i, 0)),
        ],
        out_specs=pl.BlockSpec((TB, D), lambda i: (i, 0)),
        out_shape=jax.ShapeDtypeStruct((T, D), jnp.float32),
    )(wts, z0, z1)


def _route(experts):
    e_flat = experts.reshape(-1).astype(jnp.int32)
    n = e_flat.shape[0]
    order = jnp.argsort(e_flat)
    e_sorted = e_flat[order]
    counts = jnp.zeros((E,), jnp.int32).at[e_flat].add(1)
    padded = ((counts + BM - 1) // BM) * BM
    cum_padded = jnp.cumsum(padded)
    padded_start = cum_padded - padded
    group_start = jnp.cumsum(counts) - counts
    pos = jnp.arange(n, dtype=jnp.int32)
    dest_sorted = padded_start[e_sorted] + (pos - group_start[e_sorted])
    dest = jnp.zeros((n,), jnp.int32).at[order].set(dest_sorted)
    src_token = jnp.zeros((PAD_CAP,), jnp.int32).at[dest].set(pos // K)
    total_padded = cum_padded[-1]
    num_active = total_padded // BM
    blk_start = jnp.arange(G, dtype=jnp.int32) * BM
    be_raw = jnp.searchsorted(cum_padded, blk_start, side="right").astype(jnp.int32)
    be_last = jnp.minimum(be_raw[jnp.maximum(num_active - 1, 0)], E - 1)
    block_expert = jnp.where(blk_start < total_padded,
                             jnp.minimum(be_raw, E - 1), be_last)
    return dest.reshape(T, K), src_token, block_expert, num_active.reshape(1)


def kernel(x, Wg, w1, w2, w3):
    wts, experts = _gating(x, Wg)
    dest, src_token, block_expert, num_active = _route(experts)
    x_sorted = _sc_gather(x, src_token, PAD_CAP)
    out_sorted = _gmm(x_sorted, w1, w2, w3, block_expert, num_active)
    z = _sc_gather(out_sorted, dest.T.reshape(-1), T * K)
    return _combine(wts, z[:T], z[T:])
